# baseline (speedup 1.0000x reference)
import functools

import jax
import jax.numpy as jnp
from jax import lax
from jax.experimental import pallas as pl
from jax.experimental.pallas import tpu as pltpu
from jax.experimental.pallas import tpu_sc as plsc

_N = 10000
_NP = 10240
_E = 320000
_NC, _NS = 2, 16
_NW = _NC * _NS
_EPW = 10240
_EPAD = _NW * _EPW
_SCE = 256
_NSC = _EPW // _SCE
_NG = _SCE // 16
_NEG = -1e30


def _edge_kernel_factory(c2, c):
    nwords = _NP * c

    @functools.partial(
        pl.kernel,
        mesh=plsc.VectorSubcoreMesh(
            core_axis_name="c", subcore_axis_name="s",
            num_cores=_NC, num_subcores=_NS),
        out_type=jax.ShapeDtypeStruct((_NW, _NP * c), jnp.float32),
        compiler_params=pltpu.CompilerParams(needs_layout_passes=False),
        scratch_types=(
            [pltpu.VMEM((_SCE,), jnp.int32),
             pltpu.VMEM((_SCE,), jnp.int32)]
            + [pltpu.VMEM((_SCE,), jnp.float32)] * (2 * c2)
            + [pltpu.VMEM((c2 * c * 16,), jnp.float32),
               pltpu.VMEM((nwords,), jnp.float32),
               pltpu.VMEM((32,), jnp.int32),
               pltpu.SemaphoreType.DMA]),
    )
    def ek(a_hbm, p_hbm, src_hbm, dst_hbm, w2s_hbm, out_hbm,
           src_v, dst_v, *rest):
        acols = rest[:c2]
        pcols = rest[c2:2 * c2]
        w2v = rest[2 * c2]
        acc = rest[2 * c2 + 1]
        shbuf = rest[2 * c2 + 2]
        sem = rest[2 * c2 + 3]
        wid = lax.axis_index("s") * _NC + lax.axis_index("c")
        pltpu.sync_copy(w2s_hbm, w2v)
        zeros16 = jnp.zeros((16,), jnp.int32)
        shbuf[pl.ds(0, 16)] = zeros16 - 1

        def initbody(i, carry):
            acc[pl.ds(i * 16, 16)] = jnp.full((16,), _NEG, jnp.float32)
            return carry
        lax.fori_loop(0, nwords // 16, initbody, 0)

        def superchunk(s, carry):
            base = wid * _EPW + s * _SCE
            pltpu.sync_copy(src_hbm.at[pl.ds(base, _SCE)], src_v)
            pltpu.sync_copy(dst_hbm.at[pl.ds(base, _SCE)], dst_v)
            cps = []
            for k in range(c2):
                cps.append(pltpu.async_copy(
                    a_hbm.at[pl.ds(k * _NP, _NP)].at[plsc.Indices(src_v)],
                    acols[k], sem))
                cps.append(pltpu.async_copy(
                    p_hbm.at[pl.ds(k * _NP, _NP)].at[plsc.Indices(dst_v)],
                    pcols[k], sem))
            for cp in cps:
                cp.wait()

            def group(g, gcarry):
                off = g * 16
                dstg = dst_v[pl.ds(off, 16)]
                z = []
                for k in range(c2):
                    zk = jnp.maximum(
                        acols[k][pl.ds(off, 16)] - pcols[k][pl.ds(off, 16)],
                        0.0)
                    z.append(zk)
                ts = []
                for j in range(c):
                    tj = z[0] * w2v[pl.ds(j * 16, 16)]
                    for k in range(1, c2):
                        tj = tj + z[k] * w2v[pl.ds((k * c + j) * 16, 16)]
                    ts.append(tj)

                lanes = lax.iota(jnp.int32, 16)
                cnt = jnp.zeros((16,), jnp.int32)
                for l in range(15):
                    eq = (dstg == dstg[l]) & (lanes > l)
                    cnt = cnt + eq.astype(jnp.int32)
                nrounds = jnp.max(cnt) + 1

                def body(r, bcarry):
                    m = cnt == r
                    for j in range(c):
                        idxj = dstg + j * _NP
                        old = plsc.load_gather(acc, [idxj], mask=m)
                        plsc.store_scatter(
                            acc, [idxj], jnp.maximum(old, ts[j]), mask=m)
                    return bcarry

                lax.fori_loop(0, nrounds, body, 0)
                return gcarry
            lax.fori_loop(0, _NG, group, 0)
            return carry
        lax.fori_loop(0, _NSC, superchunk, 0)
        pltpu.sync_copy(acc, out_hbm.at[wid])

    return ek


_EK = {3: _edge_kernel_factory(6, 3),
       6: _edge_kernel_factory(12, 6),
       9: _edge_kernel_factory(18, 9)}

_BLK = 1024
_GRID = _NP // _BLK


def _prep0_body(xt_ref, ga_ref, gp_ref, b1_ref, a_ref, p_ref):
    xb = xt_ref[...]
    pv = jnp.dot(gp_ref[...], xb, preferred_element_type=jnp.float32)
    a_ref[...] = jnp.dot(ga_ref[...], xb,
                         preferred_element_type=jnp.float32) + b1_ref[...]
    p_ref[...] = pv


def _prep0(xt, gat, gpt, b1t):
    c2 = gat.shape[0]
    return pl.pallas_call(
        _prep0_body,
        grid=(_GRID,),
        in_specs=[
            pl.BlockSpec((3, _BLK), lambda i: (0, i)),
            pl.BlockSpec((c2, 3), lambda i: (0, 0)),
            pl.BlockSpec((c2, 3), lambda i: (0, 0)),
            pl.BlockSpec((c2, 1), lambda i: (0, 0)),
        ],
        out_specs=[
            pl.BlockSpec((c2, _BLK), lambda i: (0, i)),
            pl.BlockSpec((c2, _BLK), lambda i: (0, i)),
        ],
        out_shape=[
            jax.ShapeDtypeStruct((c2, _NP), jnp.float32),
            jax.ShapeDtypeStruct((c2, _NP), jnp.float32),
        ],
    )(xt, gat, gpt, b1t)


def _prep_body(part_ref, xt_ref, ga_ref, gp_ref, b1_ref, b2_ref,
               a_ref, p_ref):
    h = jnp.maximum(jnp.max(part_ref[...], axis=0) + b2_ref[...], 0.0)
    xb = xt_ref[...]
    pv = jnp.dot(gp_ref[...], xb, preferred_element_type=jnp.float32)
    a_ref[...] = (jnp.dot(ga_ref[...], h, preferred_element_type=jnp.float32)
                  + pv + b1_ref[...])
    p_ref[...] = pv


def _prep(parts, xt, gat, gpt, b1t, b2t):
    cp = parts.shape[1]
    c2 = gat.shape[0]
    return pl.pallas_call(
        _prep_body,
        grid=(_GRID,),
        in_specs=[
            pl.BlockSpec((_NW, cp, _BLK), lambda i: (0, 0, i)),
            pl.BlockSpec((3, _BLK), lambda i: (0, i)),
            pl.BlockSpec((c2, cp), lambda i: (0, 0)),
            pl.BlockSpec((c2, 3), lambda i: (0, 0)),
            pl.BlockSpec((c2, 1), lambda i: (0, 0)),
            pl.BlockSpec((cp, 1), lambda i: (0, 0)),
        ],
        out_specs=[
            pl.BlockSpec((c2, _BLK), lambda i: (0, i)),
            pl.BlockSpec((c2, _BLK), lambda i: (0, i)),
        ],
        out_shape=[
            jax.ShapeDtypeStruct((c2, _NP), jnp.float32),
            jax.ShapeDtypeStruct((c2, _NP), jnp.float32),
        ],
    )(parts, xt, gat, gpt, b1t, b2t)


def _head_body(part_ref, b2_ref, fw_ref, fb_ref, cw_ref, cb_ref,
               o_ref, acc_ref):
    i = pl.program_id(0)
    h = jnp.maximum(jnp.max(part_ref[...], axis=0) + b2_ref[...], 0.0)
    bm = jnp.max(h, axis=1, keepdims=True)

    @pl.when(i == 0)
    def _():
        acc_ref[0:9, 0:1] = bm

    @pl.when(i > 0)
    def _():
        acc_ref[0:9, 0:1] = jnp.maximum(acc_ref[0:9, 0:1], bm)

    hp = acc_ref[0:9, 0:1]
    hf = jnp.maximum(
        jnp.dot(fw_ref[...], hp, preferred_element_type=jnp.float32)
        + fb_ref[...], 0.0)
    logits = (jnp.dot(cw_ref[...], hf, preferred_element_type=jnp.float32)
              + cb_ref[...])
    m = jnp.max(logits, axis=0, keepdims=True)
    e = jnp.exp(logits - m)
    o_ref[...] = e / jnp.sum(e, axis=0, keepdims=True)


def _head(parts, b2t, fwt, fbt, cwt, cbt):
    return pl.pallas_call(
        _head_body,
        grid=(_GRID,),
        in_specs=[
            pl.BlockSpec((_NW, 9, _BLK), lambda i: (0, 0, i)),
            pl.BlockSpec((9, 1), lambda i: (0, 0)),
            pl.BlockSpec((6, 9), lambda i: (0, 0)),
            pl.BlockSpec((6, 1), lambda i: (0, 0)),
            pl.BlockSpec((40, 6), lambda i: (0, 0)),
            pl.BlockSpec((40, 1), lambda i: (0, 0)),
        ],
        out_specs=pl.BlockSpec((40, 1), lambda i: (0, 0)),
        out_shape=jax.ShapeDtypeStruct((40, 1), jnp.float32),
        scratch_shapes=[pltpu.VMEM((16, 128), jnp.float32)],
    )(parts, b2t, fwt, fbt, cwt, cbt)


def _w2splat(w2):
    c2, c = w2.shape
    return jnp.broadcast_to(w2.reshape(c2 * c, 1), (c2 * c, 16)).reshape(-1)


def kernel(x, edge_index, p0_proj, p0_w1, p0_b1, p0_w2, p0_b2,
           p1_proj, p1_w1, p1_b1, p1_w2, p1_b2,
           p2_proj, p2_w1, p2_b1, p2_w2, p2_b2,
           ffw_w, ffw_b, cls_w, cls_b):
    src = edge_index[0]
    dst = edge_index[1]
    pad = _EPAD - _E
    src_p = jnp.concatenate([src, jnp.broadcast_to(src[0], (pad,))])
    dst_p = jnp.concatenate([dst, jnp.broadcast_to(dst[0], (pad,))])
    xt = jnp.pad(x.T, ((0, 0), (0, _NP - _N)))

    layers = []
    for proj, w1, b1, w2, b2 in (
            (p0_proj, p0_w1, p0_b1, p0_w2, p0_b2),
            (p1_proj, p1_w1, p1_b1, p1_w2, p1_b2),
            (p2_proj, p2_w1, p2_b1, p2_w2, p2_b2)):
        gat = (proj @ w1[:128]).T
        gpt = w1[128:].T
        layers.append((gat, gpt, b1.reshape(-1, 1), _w2splat(w2),
                       b2.reshape(-1, 1), w2.shape[1]))

    gat0, gpt0, b1t0, w2s0, b2t0, c0 = layers[0]
    a, p = _prep0(xt, gat0 + gpt0, gpt0, b1t0)
    parts = _EK[c0](a.reshape(-1), p.reshape(-1), src_p, dst_p, w2s0)
    parts = parts.reshape(_NW, c0, _NP)

    for li in (1, 2):
        gat, gpt, b1t, w2s, b2t, c = layers[li]
        b2t_prev = layers[li - 1][4]
        a, p = _prep(parts, xt, gat, gpt, b1t, b2t_prev)
        parts = _EK[c](a.reshape(-1), p.reshape(-1), src_p, dst_p, w2s)
        parts = parts.reshape(_NW, c, _NP)

    out = _head(parts, layers[2][4], ffw_w.T, ffw_b.reshape(-1, 1),
                cls_w.T, cls_b.reshape(-1, 1))
    return out.reshape(40)


# speedup vs baseline: 1.0451x; 1.0451x over previous
import functools

import jax
import jax.numpy as jnp
from jax import lax
from jax.experimental import pallas as pl
from jax.experimental.pallas import tpu as pltpu
from jax.experimental.pallas import tpu_sc as plsc

_N = 10000
_NP = 10240
_E = 320000
_NC, _NS = 2, 16
_NW = _NC * _NS
_EPW = 10240
_EPAD = _NW * _EPW
_SCE = 256
_NSC = _EPW // _SCE
_NG = _SCE // 16
_NEG = -1e30


def _edge_kernel_factory(c2, c):
    nwords = _NP * c

    @functools.partial(
        pl.kernel,
        mesh=plsc.VectorSubcoreMesh(
            core_axis_name="c", subcore_axis_name="s",
            num_cores=_NC, num_subcores=_NS),
        out_type=jax.ShapeDtypeStruct((_NW, _NP * c), jnp.float32),
        compiler_params=pltpu.CompilerParams(needs_layout_passes=False),
        scratch_types=(
            [pltpu.VMEM((_SCE,), jnp.int32),
             pltpu.VMEM((_SCE,), jnp.int32)]
            + [pltpu.VMEM((_SCE,), jnp.float32)] * (2 * c2)
            + [pltpu.VMEM((c2 * c * 16,), jnp.float32),
               pltpu.VMEM((nwords,), jnp.float32),
               pltpu.VMEM((32,), jnp.int32),
               pltpu.SemaphoreType.DMA]),
    )
    def ek(a_hbm, p_hbm, src_hbm, dst_hbm, w2s_hbm, out_hbm,
           src_v, dst_v, *rest):
        acols = rest[:c2]
        pcols = rest[c2:2 * c2]
        w2v = rest[2 * c2]
        acc = rest[2 * c2 + 1]
        shbuf = rest[2 * c2 + 2]
        sem = rest[2 * c2 + 3]
        wid = lax.axis_index("s") * _NC + lax.axis_index("c")
        pltpu.sync_copy(w2s_hbm, w2v)
        zeros16 = jnp.zeros((16,), jnp.int32)
        shbuf[pl.ds(0, 16)] = zeros16 - 1

        def initbody(i, carry):
            acc[pl.ds(i * 16, 16)] = jnp.full((16,), _NEG, jnp.float32)
            return carry
        lax.fori_loop(0, nwords // 16, initbody, 0)

        def superchunk(s, carry):
            base = wid * _EPW + s * _SCE
            pltpu.sync_copy(src_hbm.at[pl.ds(base, _SCE)], src_v)
            pltpu.sync_copy(dst_hbm.at[pl.ds(base, _SCE)], dst_v)
            cps = []
            for k in range(c2):
                cps.append(pltpu.async_copy(
                    a_hbm.at[pl.ds(k * _NP, _NP)].at[plsc.Indices(src_v)],
                    acols[k], sem))
                cps.append(pltpu.async_copy(
                    p_hbm.at[pl.ds(k * _NP, _NP)].at[plsc.Indices(dst_v)],
                    pcols[k], sem))
            for cp in cps:
                cp.wait()

            def group(g, gcarry):
                off = g * 16
                dstg = dst_v[pl.ds(off, 16)]
                z = []
                for k in range(c2):
                    zk = jnp.maximum(
                        acols[k][pl.ds(off, 16)] - pcols[k][pl.ds(off, 16)],
                        0.0)
                    z.append(zk)
                ts = []
                for j in range(c):
                    tj = z[0] * w2v[pl.ds(j * 16, 16)]
                    for k in range(1, c2):
                        tj = tj + z[k] * w2v[pl.ds((k * c + j) * 16, 16)]
                    ts.append(tj)

                lanes = lax.iota(jnp.int32, 16)
                cnt = jnp.zeros((16,), jnp.int32)
                for l in range(15):
                    eq = (dstg == dstg[l]) & (lanes > l)
                    cnt = cnt + eq.astype(jnp.int32)
                nrounds = 16

                def body(r, bcarry):
                    m = cnt == r
                    for j in range(c):
                        idxj = dstg + j * _NP
                        old = plsc.load_gather(acc, [idxj], mask=m)
                        plsc.store_scatter(
                            acc, [idxj], jnp.maximum(old, ts[j]), mask=m)
                    return bcarry

                lax.fori_loop(0, nrounds, body, 0)
                return gcarry
            lax.fori_loop(0, _NG, group, 0)
            return carry
        lax.fori_loop(0, _NSC, superchunk, 0)
        pltpu.sync_copy(acc, out_hbm.at[wid])

    return ek


_EK = {3: _edge_kernel_factory(6, 3),
       6: _edge_kernel_factory(12, 6),
       9: _edge_kernel_factory(18, 9)}

_BLK = 1024
_GRID = _NP // _BLK


def _prep0_body(xt_ref, ga_ref, gp_ref, b1_ref, a_ref, p_ref):
    xb = xt_ref[...]
    pv = jnp.dot(gp_ref[...], xb, preferred_element_type=jnp.float32)
    a_ref[...] = jnp.dot(ga_ref[...], xb,
                         preferred_element_type=jnp.float32) + b1_ref[...]
    p_ref[...] = pv


def _prep0(xt, gat, gpt, b1t):
    c2 = gat.shape[0]
    return pl.pallas_call(
        _prep0_body,
        grid=(_GRID,),
        in_specs=[
            pl.BlockSpec((3, _BLK), lambda i: (0, i)),
            pl.BlockSpec((c2, 3), lambda i: (0, 0)),
            pl.BlockSpec((c2, 3), lambda i: (0, 0)),
            pl.BlockSpec((c2, 1), lambda i: (0, 0)),
        ],
        out_specs=[
            pl.BlockSpec((c2, _BLK), lambda i: (0, i)),
            pl.BlockSpec((c2, _BLK), lambda i: (0, i)),
        ],
        out_shape=[
            jax.ShapeDtypeStruct((c2, _NP), jnp.float32),
            jax.ShapeDtypeStruct((c2, _NP), jnp.float32),
        ],
    )(xt, gat, gpt, b1t)


def _prep_body(part_ref, xt_ref, ga_ref, gp_ref, b1_ref, b2_ref,
               a_ref, p_ref):
    h = jnp.maximum(jnp.max(part_ref[...], axis=0) + b2_ref[...], 0.0)
    xb = xt_ref[...]
    pv = jnp.dot(gp_ref[...], xb, preferred_element_type=jnp.float32)
    a_ref[...] = (jnp.dot(ga_ref[...], h, preferred_element_type=jnp.float32)
                  + pv + b1_ref[...])
    p_ref[...] = pv


def _prep(parts, xt, gat, gpt, b1t, b2t):
    cp = parts.shape[1]
    c2 = gat.shape[0]
    return pl.pallas_call(
        _prep_body,
        grid=(_GRID,),
        in_specs=[
            pl.BlockSpec((_NW, cp, _BLK), lambda i: (0, 0, i)),
            pl.BlockSpec((3, _BLK), lambda i: (0, i)),
            pl.BlockSpec((c2, cp), lambda i: (0, 0)),
            pl.BlockSpec((c2, 3), lambda i: (0, 0)),
            pl.BlockSpec((c2, 1), lambda i: (0, 0)),
            pl.BlockSpec((cp, 1), lambda i: (0, 0)),
        ],
        out_specs=[
            pl.BlockSpec((c2, _BLK), lambda i: (0, i)),
            pl.BlockSpec((c2, _BLK), lambda i: (0, i)),
        ],
        out_shape=[
            jax.ShapeDtypeStruct((c2, _NP), jnp.float32),
            jax.ShapeDtypeStruct((c2, _NP), jnp.float32),
        ],
    )(parts, xt, gat, gpt, b1t, b2t)


def _head_body(part_ref, b2_ref, fw_ref, fb_ref, cw_ref, cb_ref,
               o_ref, acc_ref):
    i = pl.program_id(0)
    h = jnp.maximum(jnp.max(part_ref[...], axis=0) + b2_ref[...], 0.0)
    bm = jnp.max(h, axis=1, keepdims=True)

    @pl.when(i == 0)
    def _():
        acc_ref[0:9, 0:1] = bm

    @pl.when(i > 0)
    def _():
        acc_ref[0:9, 0:1] = jnp.maximum(acc_ref[0:9, 0:1], bm)

    hp = acc_ref[0:9, 0:1]
    hf = jnp.maximum(
        jnp.dot(fw_ref[...], hp, preferred_element_type=jnp.float32)
        + fb_ref[...], 0.0)
    logits = (jnp.dot(cw_ref[...], hf, preferred_element_type=jnp.float32)
              + cb_ref[...])
    m = jnp.max(logits, axis=0, keepdims=True)
    e = jnp.exp(logits - m)
    o_ref[...] = e / jnp.sum(e, axis=0, keepdims=True)


def _head(parts, b2t, fwt, fbt, cwt, cbt):
    return pl.pallas_call(
        _head_body,
        grid=(_GRID,),
        in_specs=[
            pl.BlockSpec((_NW, 9, _BLK), lambda i: (0, 0, i)),
            pl.BlockSpec((9, 1), lambda i: (0, 0)),
            pl.BlockSpec((6, 9), lambda i: (0, 0)),
            pl.BlockSpec((6, 1), lambda i: (0, 0)),
            pl.BlockSpec((40, 6), lambda i: (0, 0)),
            pl.BlockSpec((40, 1), lambda i: (0, 0)),
        ],
        out_specs=pl.BlockSpec((40, 1), lambda i: (0, 0)),
        out_shape=jax.ShapeDtypeStruct((40, 1), jnp.float32),
        scratch_shapes=[pltpu.VMEM((16, 128), jnp.float32)],
    )(parts, b2t, fwt, fbt, cwt, cbt)


def _w2splat(w2):
    c2, c = w2.shape
    return jnp.broadcast_to(w2.reshape(c2 * c, 1), (c2 * c, 16)).reshape(-1)


def kernel(x, edge_index, p0_proj, p0_w1, p0_b1, p0_w2, p0_b2,
           p1_proj, p1_w1, p1_b1, p1_w2, p1_b2,
           p2_proj, p2_w1, p2_b1, p2_w2, p2_b2,
           ffw_w, ffw_b, cls_w, cls_b):
    src = edge_index[0]
    dst = edge_index[1]
    pad = _EPAD - _E
    src_p = jnp.concatenate([src, jnp.broadcast_to(src[0], (pad,))])
    dst_p = jnp.concatenate([dst, jnp.broadcast_to(dst[0], (pad,))])
    xt = jnp.pad(x.T, ((0, 0), (0, _NP - _N)))

    layers = []
    for proj, w1, b1, w2, b2 in (
            (p0_proj, p0_w1, p0_b1, p0_w2, p0_b2),
            (p1_proj, p1_w1, p1_b1, p1_w2, p1_b2),
            (p2_proj, p2_w1, p2_b1, p2_w2, p2_b2)):
        gat = (proj @ w1[:128]).T
        gpt = w1[128:].T
        layers.append((gat, gpt, b1.reshape(-1, 1), _w2splat(w2),
                       b2.reshape(-1, 1), w2.shape[1]))

    gat0, gpt0, b1t0, w2s0, b2t0, c0 = layers[0]
    a, p = _prep0(xt, gat0 + gpt0, gpt0, b1t0)
    parts = _EK[c0](a.reshape(-1), p.reshape(-1), src_p, dst_p, w2s0)
    parts = parts.reshape(_NW, c0, _NP)

    for li in (1, 2):
        gat, gpt, b1t, w2s, b2t, c = layers[li]
        b2t_prev = layers[li - 1][4]
        a, p = _prep(parts, xt, gat, gpt, b1t, b2t_prev)
        parts = _EK[c](a.reshape(-1), p.reshape(-1), src_p, dst_p, w2s)
        parts = parts.reshape(_NW, c, _NP)

    out = _head(parts, layers[2][4], ffw_w.T, ffw_b.reshape(-1, 1),
                cls_w.T, cls_b.reshape(-1, 1))
    return out.reshape(40)
